# baseline (speedup 1.0000x reference)
import functools

import jax
import jax.numpy as jnp
from jax import lax
from jax.experimental import pallas as pl
from jax.experimental.pallas import tpu as pltpu
from jax.experimental.pallas import tpu_sc as plsc

_NC = 2
_NS = 16
_L = 16
_CH = 32


def _embed_stem(idx_flat, tok_emb, pos):
    BT = idx_flat.shape[0]
    T, D = pos.shape
    B = BT // T
    NW = _NC * _NS
    TW = T // NW
    NCHUNK = B * TW // _CH

    mesh = plsc.VectorSubcoreMesh(core_axis_name="c", subcore_axis_name="s")

    @functools.partial(
        pl.kernel,
        mesh=mesh,
        out_type=jax.ShapeDtypeStruct((BT, D), jnp.float32),
        scratch_types=[
            pltpu.VMEM((B, TW), jnp.int32),
            pltpu.VMEM((TW, D), jnp.float32),
            pltpu.VMEM((2, _CH, D), jnp.float32),
            pltpu.SemaphoreType.DMA,
            pltpu.SemaphoreType.DMA,
            pltpu.SemaphoreType.DMA,
            pltpu.SemaphoreType.DMA,
            pltpu.SemaphoreType.DMA,
        ],
    )
    def k(idx_hbm, tab_hbm, pos_hbm, out_hbm, idx_v, pos_v, buf, psem,
          gsem0, gsem1, wsem0, wsem1):
        gsem = (gsem0, gsem1)
        wsem = (wsem0, wsem1)
        wid = lax.axis_index("s") * _NC + lax.axis_index("c")
        t0 = wid * TW
        for b in range(B):
            pltpu.sync_copy(idx_hbm.at[pl.ds(b * T + t0, TW)], idx_v.at[b])
        pos_cp = pltpu.async_copy(pos_hbm.at[pl.ds(t0, TW)], pos_v, psem)

        def chunk_gather(kk):
            b, h = kk // (TW // _CH), kk % (TW // _CH)
            return pltpu.async_copy(
                tab_hbm.at[idx_v.at[b, pl.ds(h * _CH, _CH)]],
                buf.at[kk % 2], gsem[kk % 2])

        gathers = {0: chunk_gather(0)}
        writes = {}
        pos_cp.wait()
        for kk in range(NCHUNK):
            p = kk % 2
            if kk + 1 < NCHUNK:
                if kk - 1 in writes:
                    writes.pop(kk - 1).wait()
                gathers[kk + 1] = chunk_gather(kk + 1)
            gathers.pop(kk).wait()
            b, h = kk // (TW // _CH), kk % (TW // _CH)

            def row_add(r, _):
                for c in range(D // _L):
                    sl = pl.ds(c * _L, _L)
                    buf[p, r, sl] = buf[p, r, sl] + pos_v[h * _CH + r, sl]
                return 0

            lax.fori_loop(0, _CH, row_add, 0)
            writes[kk] = pltpu.async_copy(
                buf.at[p], out_hbm.at[pl.ds(b * T + t0 + h * _CH, _CH)],
                wsem[p])
        for kk in sorted(writes):
            writes.pop(kk).wait()

    return k(idx_flat, tok_emb, pos)


def kernel(idx, tok_emb, pos_embed):
    b, t = idx.shape
    d = tok_emb.shape[1]
    idx_flat = idx.reshape(-1).astype(jnp.int32)
    pos = pos_embed[0, :t, :]
    out = _embed_stem(idx_flat, tok_emb, pos)
    return out.reshape(b, t, d)
